# baseline (device time: 40120 ns/iter reference)
import os

import jax
import jax.numpy as jnp
from jax import lax
from jax.experimental import pallas as pl
from jax.experimental.pallas import tpu as pltpu

N_DEV = 16
NG = 2
NSC = int(os.environ.get("KERNEL_NSC", "4"))
NU = NG * NSC
NT = 6
NSLOT = 10
_NOCOMM = os.environ.get("KERNEL_NOCOMM") == "1"


def kernel(x, w_mat):
    m, _ = x.shape
    _, n = w_mat.shape
    cw = n // NU
    h2, h4 = m // 2, m // 4
    ch = m // N_DEV

    def body(x_ref, w_ref, out_ref, acc, rb0, rb1, rbz, send_sems, recv_sems):
        p = lax.axis_index("i")
        q = lax.rem(p, 4)
        z = lax.div(p, 4)
        dims = {
            "x": (lax.rem(q, 2) ^ lax.div(q, 2), p + 1 - 2 * lax.rem(q, 2)),
            "y": (lax.div(q, 2), p + 3 - 2 * q),
        }
        zpeers = [lax.rem(p + 4 * d, N_DEV) for d in (1, 2, 3)]
        orders = (("x", "y"), ("y", "x"))

        partners = (dims["x"][1], dims["y"][1], *zpeers)
        barrier = pltpu.get_barrier_semaphore()
        for nbr in partners:
            pl.semaphore_signal(
                barrier, inc=1, device_id=(nbr,),
                device_id_type=pl.DeviceIdType.MESH,
            )
        pl.semaphore_wait(barrier, len(partners))

        def gemm_unit(u):
            co = pl.ds(u * cw, cw)
            acc[:, co] = jnp.dot(
                x_ref[:, :], w_ref[:, co], preferred_element_type=jnp.float32
            ).astype(jnp.bfloat16)

        geo = []
        for g in range(NG):
            b0, p0 = dims[orders[g][0]]
            b1, p1 = dims[orders[g][1]]
            o1 = b0 * h2
            geo.append((
                (1 - b0) * h2, o1, o1 + (1 - b1) * h4, o1 + b1 * h4, p0, p1,
            ))

        rdmas = {}

        def rc(u, slot, src, dst, peer):
            d = pltpu.make_async_remote_copy(
                src_ref=src, dst_ref=dst,
                send_sem=send_sems.at[u, slot], recv_sem=recv_sems.at[u, slot],
                device_id=(peer,), device_id_type=pl.DeviceIdType.MESH,
            )
            d.start()
            return d

        def issue(u, k):
            s0, o1, s1, oxy, p0, p1 = geo[u // NSC]
            co = pl.ds(u * cw, cw)
            my_chunk = pl.ds(oxy + z * ch, ch)
            if k == 0:
                gemm_unit(u)
                src = acc.at[pl.ds(s0, h2), co]
                rdmas[(u, k)] = [rc(u, 0, src, rb0.at[u], p0)]
            elif k == 1:
                src = acc.at[pl.ds(s1, h4), co]
                rdmas[(u, k)] = [rc(u, 1, src, rb1.at[u], p1)]
            elif k == 2:
                ds_ = []
                for d in (1, 2, 3):
                    cz = lax.rem(z + d, 4)
                    src = acc.at[pl.ds(oxy + cz * ch, ch), co]
                    ds_.append(rc(u, 1 + d, src, rbz.at[u, d - 1], zpeers[d - 1]))
                rdmas[(u, k)] = ds_
            elif k == 3:
                src = acc.at[my_chunk, co]
                rdmas[(u, k)] = [
                    rc(u, 4 + d, src, src, zpeers[d - 1]) for d in (1, 2, 3)
                ]
            elif k == 4:
                src = acc.at[pl.ds(oxy, h4), co]
                rdmas[(u, k)] = [rc(u, 8, src, src, p1)]
            else:
                src = acc.at[pl.ds(o1, h2), co]
                rdmas[(u, k)] = [rc(u, 9, src, src, p0)]

        def apply(u, k):
            for d in rdmas.pop((u, k)):
                d.wait()
            s0, o1, s1, oxy, _, _ = geo[u // NSC]
            co = pl.ds(u * cw, cw)
            if k == 0:
                rows = pl.ds(o1, h2)
                acc[rows, co] = acc[rows, co] + rb0[u]
            elif k == 1:
                rows = pl.ds(oxy, h4)
                acc[rows, co] = acc[rows, co] + rb1[u]
            elif k == 2:
                rows = pl.ds(oxy + z * ch, ch)
                acc[rows, co] = (
                    acc[rows, co] + rbz[u, 0] + rbz[u, 1] + rbz[u, 2]
                )

        for t in range(NSC + NT):
            for u in range(NU):
                k = t - (u % NSC)
                if not _NOCOMM:
                    if 0 < k <= NT:
                        apply(u, k - 1)
                if k == NT:
                    co = pl.ds(u * cw, cw)
                    out_ref[:, co] = acc[:, co].astype(jnp.float32)
                if not _NOCOMM:
                    if 0 <= k < NT:
                        issue(u, k)

    return pl.pallas_call(
        body,
        out_shape=jax.ShapeDtypeStruct((m, n), jnp.float32),
        in_specs=[
            pl.BlockSpec(memory_space=pltpu.VMEM),
            pl.BlockSpec(memory_space=pltpu.VMEM),
        ],
        out_specs=pl.BlockSpec(memory_space=pltpu.VMEM),
        scratch_shapes=[
            pltpu.VMEM((m, n), jnp.bfloat16),
            pltpu.VMEM((NU, h2, cw), jnp.bfloat16),
            pltpu.VMEM((NU, h4, cw), jnp.bfloat16),
            pltpu.VMEM((NU, 3, ch, cw), jnp.bfloat16),
            pltpu.SemaphoreType.DMA((NU, NSLOT)),
            pltpu.SemaphoreType.DMA((NU, NSLOT)),
        ],
        compiler_params=pltpu.CompilerParams(collective_id=0),
    )(x, w_mat)
